# Initial kernel scaffold; baseline (speedup 1.0000x reference)
import jax
import jax.numpy as jnp
from jax.experimental import pallas as pl


def kernel(x, edge_index, pos_src_idx, pos_dst_idx, neg_src_idx, neg_dst_idx, W_self1, W_neigh1, b1, W_self2, W_neigh2, b2):
    raise NotImplementedError("write your pallas kernel here")


# speedup vs baseline: 5.6515x; 5.6515x over previous
import jax
import jax.numpy as jnp
from jax import lax
from jax.experimental import pallas as pl
from jax.experimental.pallas import tpu as pltpu
import jax.experimental.pallas.tpu_sc as plsc

N = 10000
D = 128
E = 320000
B = 4096

NC = 2
NS = 16
NW = NC * NS
LANES = 16

K = 128
CHUNKS = E // K
ITERS = -(-CHUNKS // NW)
FR = 80
FCH = N // FR
FITERS = -(-FCH // NS)
DEGW = 128

TB = 4 * B
GPW = TB // NW
GCH = GPW // K


def _mesh():
    return plsc.VectorSubcoreMesh(core_axis_name="c", subcore_axis_name="s",
                                  num_cores=NC, num_subcores=NS)


def _zero_2d(ref, rows, width):
    zero = jnp.zeros((LANES,), jnp.float32)

    def body(i, carry):
        for cb in range(width // LANES):
            ref[i, pl.ds(cb * LANES, LANES)] = zero
        return carry

    lax.fori_loop(0, rows, body, 0)


def _fill_ones(ref, rows, width):
    one = jnp.ones((LANES,), jnp.float32)

    def body(i, carry):
        for cb in range(width // LANES):
            ref[i, pl.ds(cb * LANES, LANES)] = one
        return carry

    lax.fori_loop(0, rows, body, 0)


def _sc_segsum_body(y_hbm, src_hbm, dst_hbm, za_out, zb_out,
                    src_v, dst_v, rows_v, zrow_v, acc_s, sem):
    c = lax.axis_index("c")
    s = lax.axis_index("s")
    wid = s * NC + c

    _zero_2d(zrow_v, FR, D)
    for j in range(FITERS):
        fid = s + j * NS

        @pl.when(fid < FCH)
        def _():
            r0 = pl.multiple_of(fid * FR, 8)
            pltpu.sync_copy(zrow_v, acc_s.at[pl.ds(r0, FR)])
    plsc.subcore_barrier()

    def body(i, carry):
        cid = wid + i * NW

        @pl.when(cid < CHUNKS)
        def _():
            base = pl.multiple_of(cid * K, 8)
            pltpu.sync_copy(src_hbm.at[pl.ds(base, K)], src_v)
            pltpu.sync_copy(dst_hbm.at[pl.ds(base, K)], dst_v)
            pltpu.async_copy(y_hbm.at[src_v], rows_v, sem).wait()
            pltpu.sync_copy(rows_v, acc_s.at[dst_v], add=True)

        return carry

    lax.fori_loop(0, ITERS, body, 0)
    plsc.subcore_barrier()

    for j in range(FITERS):
        fid = s + j * NS

        @pl.when(fid < FCH)
        def _():
            r0 = pl.multiple_of(fid * FR, 8)

            @pl.when(c == 0)
            def _():
                pltpu.sync_copy(acc_s.at[pl.ds(r0, FR)],
                                za_out.at[pl.ds(r0, FR)])

            @pl.when(c == 1)
            def _():
                pltpu.sync_copy(acc_s.at[pl.ds(r0, FR)],
                                zb_out.at[pl.ds(r0, FR)])


def _sc_segsum(y, src, dst):
    zshape = jax.ShapeDtypeStruct((N, D), jnp.float32)
    k = pl.kernel(
        _sc_segsum_body,
        out_type=[zshape, zshape],
        mesh=_mesh(),
        scratch_types=[
            pltpu.VMEM((K,), jnp.int32),
            pltpu.VMEM((K,), jnp.int32),
            pltpu.VMEM((K, D), jnp.float32),
            pltpu.VMEM((FR, D), jnp.float32),
            pltpu.VMEM_SHARED((N, D), jnp.float32),
            pltpu.SemaphoreType.DMA,
        ],
    )
    return k(y, src, dst)


def _sc_deg_body(dst_hbm, dega_out, degb_out,
                 dst_v, ones_v, zdeg_v, deg_s):
    c = lax.axis_index("c")
    s = lax.axis_index("s")
    wid = s * NC + c

    _zero_2d(zdeg_v, FR, DEGW)
    _fill_ones(ones_v, K, DEGW)
    for j in range(FITERS):
        fid = s + j * NS

        @pl.when(fid < FCH)
        def _():
            r0 = pl.multiple_of(fid * FR, 8)
            pltpu.sync_copy(zdeg_v, deg_s.at[pl.ds(r0, FR)])
    plsc.subcore_barrier()

    def body(i, carry):
        cid = wid + i * NW

        @pl.when(cid < CHUNKS)
        def _():
            base = pl.multiple_of(cid * K, 8)
            pltpu.sync_copy(dst_hbm.at[pl.ds(base, K)], dst_v)
            pltpu.sync_copy(ones_v, deg_s.at[dst_v], add=True)

        return carry

    lax.fori_loop(0, ITERS, body, 0)
    plsc.subcore_barrier()

    for j in range(FITERS):
        fid = s + j * NS

        @pl.when(fid < FCH)
        def _():
            r0 = pl.multiple_of(fid * FR, 8)

            @pl.when(c == 0)
            def _():
                pltpu.sync_copy(deg_s.at[pl.ds(r0, FR)],
                                dega_out.at[pl.ds(r0, FR)])

            @pl.when(c == 1)
            def _():
                pltpu.sync_copy(deg_s.at[pl.ds(r0, FR)],
                                degb_out.at[pl.ds(r0, FR)])


def _sc_deg(dst):
    dshape = jax.ShapeDtypeStruct((N, DEGW), jnp.float32)
    k = pl.kernel(
        _sc_deg_body,
        out_type=[dshape, dshape],
        mesh=_mesh(),
        scratch_types=[
            pltpu.VMEM((K,), jnp.int32),
            pltpu.VMEM((K, DEGW), jnp.float32),
            pltpu.VMEM((FR, DEGW), jnp.float32),
            pltpu.VMEM_SHARED((N, DEGW), jnp.float32),
        ],
    )
    return k(dst)


def _sc_gather_body(h_hbm, idx_hbm, out_hbm, idx_v, rows_v, sem):
    c = lax.axis_index("c")
    s = lax.axis_index("s")
    wid = s * NC + c
    for j in range(GCH):
        base = pl.multiple_of(wid * GPW + j * K, 8)
        pltpu.sync_copy(idx_hbm.at[pl.ds(base, K)], idx_v)
        pltpu.async_copy(h_hbm.at[idx_v], rows_v, sem).wait()
        pltpu.sync_copy(rows_v, out_hbm.at[pl.ds(base, K)])


def _sc_gather(h, idx):
    k = pl.kernel(
        _sc_gather_body,
        out_type=jax.ShapeDtypeStruct((TB, D), jnp.float32),
        mesh=_mesh(),
        scratch_types=[
            pltpu.VMEM((K,), jnp.int32),
            pltpu.VMEM((K, D), jnp.float32),
            pltpu.SemaphoreType.DMA,
        ],
    )
    return k(h, idx)


TC_R = 1000


def _tc1_body(x_ref, ws_ref, wn_ref, b_ref, a1_ref, y1_ref):
    xv = x_ref[...]
    a1_ref[...] = (jnp.dot(xv, ws_ref[...], preferred_element_type=jnp.float32)
                   + b_ref[...])
    y1_ref[...] = jnp.dot(xv, wn_ref[...], preferred_element_type=jnp.float32)


def _tc1(x, ws, wn, b):
    row_spec = pl.BlockSpec((TC_R, D), lambda i: (i, 0))
    w_spec = pl.BlockSpec((D, D), lambda i: (0, 0))
    b_spec = pl.BlockSpec((1, D), lambda i: (0, 0))
    return pl.pallas_call(
        _tc1_body,
        grid=(N // TC_R,),
        in_specs=[row_spec, w_spec, w_spec, b_spec],
        out_specs=[row_spec, row_spec],
        out_shape=[jax.ShapeDtypeStruct((N, D), jnp.float32)] * 2,
    )(x, ws, wn, b.reshape(1, D))


def _tc2_body(a1_ref, za_ref, zb_ref, da_ref, db_ref, ws_ref, wn_ref, b_ref,
              a2_ref, y2_ref):
    deg = da_ref[...] + db_ref[...]
    inv = 1.0 / jnp.maximum(deg[:, 0:1], 1.0)
    h1 = jnp.maximum(a1_ref[...] + (za_ref[...] + zb_ref[...]) * inv, 0.0)
    a2_ref[...] = (jnp.dot(h1, ws_ref[...], preferred_element_type=jnp.float32)
                   + b_ref[...])
    y2_ref[...] = jnp.dot(h1, wn_ref[...], preferred_element_type=jnp.float32)


def _tc2(a1, za, zb, da, db, ws, wn, b):
    row_spec = pl.BlockSpec((TC_R, D), lambda i: (i, 0))
    deg_spec = pl.BlockSpec((TC_R, DEGW), lambda i: (i, 0))
    w_spec = pl.BlockSpec((D, D), lambda i: (0, 0))
    b_spec = pl.BlockSpec((1, D), lambda i: (0, 0))
    return pl.pallas_call(
        _tc2_body,
        grid=(N // TC_R,),
        in_specs=[row_spec, row_spec, row_spec, deg_spec, deg_spec,
                  w_spec, w_spec, b_spec],
        out_specs=[row_spec, row_spec],
        out_shape=[jax.ShapeDtypeStruct((N, D), jnp.float32)] * 2,
    )(a1, za, zb, da, db, ws, wn, b.reshape(1, D))


def _tc3_body(a2_ref, za_ref, zb_ref, da_ref, db_ref, h2_ref):
    deg = da_ref[...] + db_ref[...]
    inv = 1.0 / jnp.maximum(deg[:, 0:1], 1.0)
    h2_ref[...] = a2_ref[...] + (za_ref[...] + zb_ref[...]) * inv


def _tc3(a2, za, zb, da, db):
    row_spec = pl.BlockSpec((TC_R, D), lambda i: (i, 0))
    deg_spec = pl.BlockSpec((TC_R, DEGW), lambda i: (i, 0))
    return pl.pallas_call(
        _tc3_body,
        grid=(N // TC_R,),
        in_specs=[row_spec, row_spec, row_spec, deg_spec, deg_spec],
        out_specs=row_spec,
        out_shape=jax.ShapeDtypeStruct((N, D), jnp.float32),
    )(a2, za, zb, da, db)


def kernel(x, edge_index, pos_src_idx, pos_dst_idx, neg_src_idx, neg_dst_idx,
           W_self1, W_neigh1, b1, W_self2, W_neigh2, b2):
    src = edge_index[0]
    dst = edge_index[1]

    dega, degb = _sc_deg(dst)
    a1, y1 = _tc1(x, W_self1, W_neigh1, b1)
    z1a, z1b = _sc_segsum(y1, src, dst)
    a2, y2 = _tc2(a1, z1a, z1b, dega, degb, W_self2, W_neigh2, b2)
    z2a, z2b = _sc_segsum(y2, src, dst)
    h2 = _tc3(a2, z2a, z2b, dega, degb)

    cat_idx = jnp.concatenate(
        [pos_src_idx, pos_dst_idx, neg_src_idx, neg_dst_idx])
    out = _sc_gather(h2, cat_idx)
    return (out[0:B], out[B:2 * B], out[2 * B:3 * B], out[3 * B:4 * B])
